# baseline (speedup 1.0000x reference)
import functools
import jax
import jax.numpy as jnp
from jax import lax
from jax.experimental import pallas as pl
from jax.experimental.pallas import tpu as pltpu
from jax.experimental.pallas import tpu_sc as plsc

N_NODES = 10000
N_EDGES = 320000
N_GRAPHS = 64
DIM = 128
HALF = 64
DIM_OUT = 10
NP = 10240

NC, NS = 2, 16
NW = NC * NS
K = 128
NCHUNK = 79
EPW = NCHUNK * K
EPAD = NW * EPW - N_EDGES
STRIPE = NP // NS

RB = 1024
NRB = NP // RB


def _make_mesh():
    return plsc.VectorSubcoreMesh(
        core_axis_name="c", subcore_axis_name="s", num_cores=NC,
        num_subcores=NS,
    )


def _deg_kernel_body(dst_hbm, out_hbm, dst_v, hist_v):
    c = lax.axis_index("c")
    s = lax.axis_index("s")
    wid = c * NS + s
    pltpu.sync_copy(dst_hbm.at[wid], dst_v)
    zero16 = jnp.zeros((16,), jnp.float32)

    @pl.loop(0, NP // 16)
    def _zero(r):
        hist_v[pl.ds(r * 16, 16)] = zero16

    one16 = jnp.ones((16,), jnp.float32)

    @pl.loop(0, NCHUNK)
    def _hist(r):
        for cc in range(K // 16):
            idx = dst_v[r, pl.ds(cc * 16, 16)]
            plsc.addupdate_scatter(hist_v, [idx], one16)

    pltpu.sync_copy(hist_v, out_hbm.at[wid, 0])


def _scatter_kernel_body(y_hbm, src_hbm, dst_hbm, out_hbm,
                         src_v, dst_v, rows_v, acc_sh, sem_a):
    c = lax.axis_index("c")
    s = lax.axis_index("s")
    wid = c * NS + s
    pltpu.sync_copy(src_hbm.at[wid], src_v)
    pltpu.sync_copy(dst_hbm.at[wid], dst_v)
    pltpu.sync_copy(
        y_hbm.at[pl.ds(s * STRIPE, STRIPE)],
        acc_sh.at[pl.ds(s * STRIPE, STRIPE)],
    )
    plsc.subcore_barrier()

    def body(j, carry):
        pltpu.async_copy(y_hbm.at[src_v.at[j]], rows_v, sem_a).wait()
        pltpu.sync_copy(rows_v, acc_sh.at[dst_v.at[j]], add=True)
        return carry

    lax.fori_loop(0, NCHUNK, body, 0)
    plsc.subcore_barrier()
    pltpu.sync_copy(
        acc_sh.at[pl.ds(s * STRIPE, STRIPE)],
        out_hbm.at[c, pl.ds(s * STRIPE, STRIPE)],
    )


def _dis_from(deg_ref):
    d2 = deg_ref[:, 0, 0, :]
    ones_nw = jnp.ones((NW, 1), jnp.float32)
    cnt = lax.dot_general(d2, ones_nw, (((0,), (0,)), ((), ())),
                          preferred_element_type=jnp.float32) + 1.0
    return lax.rsqrt(cnt)


def _ka_body(deg_ref, x_ref, w_ref, y_ref):
    dis = _dis_from(deg_ref)
    xw = jnp.dot(x_ref[...], w_ref[...], preferred_element_type=jnp.float32)
    y_ref[...] = xw * dis


_ka_call = pl.pallas_call(
    _ka_body,
    grid=(NRB,),
    in_specs=[
        pl.BlockSpec((NW, 1, 1, RB), lambda i: (0, i, 0, 0)),
        pl.BlockSpec((RB, DIM), lambda i: (i, 0)),
        pl.BlockSpec((DIM, DIM), lambda i: (0, 0)),
    ],
    out_specs=pl.BlockSpec((RB, DIM), lambda i: (i, 0)),
    out_shape=jax.ShapeDtypeStruct((NP, DIM), jnp.float32),
)


def _kb_body(deg_ref, y1_ref, acc_ref, w2_ref, b1_ref, h1_ref, y2_ref):
    dis = _dis_from(deg_ref)
    h1 = dis * (acc_ref[0] + acc_ref[1] - y1_ref[...]) + b1_ref[...]
    h1_ref[...] = h1
    y2_ref[...] = (
        jnp.dot(h1, w2_ref[...], preferred_element_type=jnp.float32) * dis
    )


_kb_call = pl.pallas_call(
    _kb_body,
    grid=(NRB,),
    in_specs=[
        pl.BlockSpec((NW, 1, 1, RB), lambda i: (0, i, 0, 0)),
        pl.BlockSpec((RB, DIM), lambda i: (i, 0)),
        pl.BlockSpec((NC, RB, DIM), lambda i: (0, i, 0)),
        pl.BlockSpec((DIM, DIM), lambda i: (0, 0)),
        pl.BlockSpec((1, DIM), lambda i: (0, 0)),
    ],
    out_specs=[
        pl.BlockSpec((RB, DIM), lambda i: (i, 0)),
        pl.BlockSpec((RB, DIM), lambda i: (i, 0)),
    ],
    out_shape=[
        jax.ShapeDtypeStruct((NP, DIM), jnp.float32),
        jax.ShapeDtypeStruct((NP, DIM), jnp.float32),
    ],
)


def _kc_body(deg_ref, y2_ref, acc_ref, b2_ref, h1_ref, batch_ref,
             wl1_ref, bl1_ref, wl2_ref, bl2_ref, out1_ref, out2_ref, p_scr):
    i = pl.program_id(0)
    dis = _dis_from(deg_ref)
    h2 = dis * (acc_ref[0] + acc_ref[1] - y2_ref[...]) + b2_ref[...]
    gid = lax.broadcasted_iota(jnp.int32, (N_GRAPHS, RB), 0)
    m = (gid == batch_ref[0]).astype(jnp.float32)
    pp1 = jnp.dot(m, h1_ref[...], preferred_element_type=jnp.float32)
    pp2 = jnp.dot(m, h2, preferred_element_type=jnp.float32)

    @pl.when(i == 0)
    def _():
        p_scr[:, :DIM] = pp1
        p_scr[:, DIM:] = pp2

    @pl.when(i > 0)
    def _():
        p_scr[:, :DIM] += pp1
        p_scr[:, DIM:] += pp2

    @pl.when(i == NRB - 1)
    def _():
        p = p_scr[...]
        t = jnp.dot(p, wl1_ref[...], preferred_element_type=jnp.float32)
        t = jnp.maximum(t + bl1_ref[...], 0.0)
        logits = (
            jnp.dot(t, wl2_ref[...], preferred_element_type=jnp.float32)
            + bl2_ref[...]
        )
        out1_ref[...] = logits
        col = lax.broadcasted_iota(jnp.int32, (N_GRAPHS, DIM), 1)
        neg = jnp.where(col < DIM_OUT, logits, jnp.float32(-1e30))
        mx = jnp.max(neg, axis=1, keepdims=True)
        lse = jnp.log(jnp.sum(jnp.exp(neg - mx), axis=1, keepdims=True)) + mx
        out2_ref[...] = logits - lse


_kc_call = pl.pallas_call(
    _kc_body,
    grid=(NRB,),
    in_specs=[
        pl.BlockSpec((NW, 1, 1, RB), lambda i: (0, i, 0, 0)),
        pl.BlockSpec((RB, DIM), lambda i: (i, 0)),
        pl.BlockSpec((NC, RB, DIM), lambda i: (0, i, 0)),
        pl.BlockSpec((1, DIM), lambda i: (0, 0)),
        pl.BlockSpec((RB, DIM), lambda i: (i, 0)),
        pl.BlockSpec((1, 1, RB), lambda i: (i, 0, 0)),
        pl.BlockSpec((2 * DIM, 2 * DIM), lambda i: (0, 0)),
        pl.BlockSpec((1, 2 * DIM), lambda i: (0, 0)),
        pl.BlockSpec((2 * DIM, DIM), lambda i: (0, 0)),
        pl.BlockSpec((1, DIM), lambda i: (0, 0)),
    ],
    out_specs=[
        pl.BlockSpec((N_GRAPHS, DIM), lambda i: (0, 0)),
        pl.BlockSpec((N_GRAPHS, DIM), lambda i: (0, 0)),
    ],
    out_shape=[
        jax.ShapeDtypeStruct((N_GRAPHS, DIM), jnp.float32),
        jax.ShapeDtypeStruct((N_GRAPHS, DIM), jnp.float32),
    ],
    scratch_shapes=[pltpu.VMEM((N_GRAPHS, 2 * DIM), jnp.float32)],
)


def kernel(x, edge_index, batch, W1, b1, W2, b2, Wl1, bl1, Wl2, bl2):
    pad = jnp.full((EPAD,), NP - 1, jnp.int32)
    src = (jnp.concatenate([edge_index[0].astype(jnp.int32), pad])
           .reshape(NW, NCHUNK, K))
    dst = (jnp.concatenate([edge_index[1].astype(jnp.int32), pad])
           .reshape(NW, NCHUNK, K))
    xp = jnp.zeros((NP, DIM), jnp.float32).at[:N_NODES].set(x)
    batch2d = (jnp.full((NP,), N_GRAPHS, jnp.int32)
               .at[:N_NODES].set(batch.astype(jnp.int32)).reshape(NRB, 1, RB))
    wl2p = jnp.zeros((2 * DIM, DIM), jnp.float32).at[:, :DIM_OUT].set(Wl2)
    bl2p = jnp.zeros((1, DIM), jnp.float32).at[0, :DIM_OUT].set(bl2)
    b1r = b1.reshape(1, DIM)
    b2r = b2.reshape(1, DIM)
    bl1r = bl1.reshape(1, 2 * DIM)

    deg_kernel = pl.kernel(
        _deg_kernel_body,
        out_type=jax.ShapeDtypeStruct((NW, 1, NP), jnp.float32),
        mesh=_make_mesh(),
        scratch_types=[
            pltpu.VMEM((NCHUNK, K), jnp.int32),
            pltpu.VMEM((NP,), jnp.float32),
        ],
        compiler_params=pltpu.CompilerParams(needs_layout_passes=False),
    )
    scatter_kernel = pl.kernel(
        _scatter_kernel_body,
        out_type=jax.ShapeDtypeStruct((NC, NP, DIM), jnp.float32),
        mesh=_make_mesh(),
        scratch_types=[
            pltpu.VMEM((NCHUNK, K), jnp.int32),
            pltpu.VMEM((NCHUNK, K), jnp.int32),
            pltpu.VMEM((K, DIM), jnp.float32),
            pltpu.VMEM_SHARED((NP, DIM), jnp.float32),
            pltpu.SemaphoreType.DMA,
        ],
    )

    deg16 = deg_kernel(dst).reshape(NW, NRB, 1, RB)
    y1 = _ka_call(deg16, xp, W1)
    acc1 = scatter_kernel(y1, src, dst)
    h1, y2 = _kb_call(deg16, y1, acc1, W2, b1r)
    acc2 = scatter_kernel(y2, src, dst)
    logits, logp = _kc_call(deg16, y2, acc2, b2r, h1, batch2d,
                            Wl1, bl1r, wl2p, bl2p)
    return (logits[:, :DIM_OUT], logp[:, :DIM_OUT])


# speedup vs baseline: 1.4665x; 1.4665x over previous
import functools
import jax
import jax.numpy as jnp
from jax import lax
from jax.experimental import pallas as pl
from jax.experimental.pallas import tpu as pltpu
from jax.experimental.pallas import tpu_sc as plsc

N_NODES = 10000
N_EDGES = 320000
N_GRAPHS = 64
DIM = 128
HALF = 64
DIM_OUT = 10
NP = 10240

NC, NS = 2, 16
NW = NC * NS
EPW = N_EDGES // NW
K = 80
NCHUNK = EPW // K
STRIPE = NP // NS

RB = 1024
NRB = NP // RB


def _make_mesh():
    return plsc.VectorSubcoreMesh(
        core_axis_name="c", subcore_axis_name="s", num_cores=NC,
        num_subcores=NS,
    )


def _deg_kernel_body(dst_hbm, out_hbm, dst_v, hist_v):
    c = lax.axis_index("c")
    s = lax.axis_index("s")
    wid = c * NS + s
    pltpu.sync_copy(dst_hbm.at[wid], dst_v)
    zero16 = jnp.zeros((16,), jnp.float32)

    @pl.loop(0, NP // 16)
    def _zero(r):
        hist_v[pl.ds(r * 16, 16)] = zero16

    one16 = jnp.ones((16,), jnp.float32)

    @pl.loop(0, NCHUNK)
    def _hist(r):
        for cc in range(K // 16):
            idx = dst_v[r, pl.ds(cc * 16, 16)]
            plsc.addupdate_scatter(hist_v, [idx], one16)

    pltpu.sync_copy(hist_v, out_hbm.at[wid, 0])


def _scatter_kernel_body(y_hbm, src_hbm, dst_hbm, out_hbm,
                         src_v, dst_v, rows_v, acc_sh, sem_a):
    c = lax.axis_index("c")
    s = lax.axis_index("s")
    wid = c * NS + s
    pltpu.sync_copy(src_hbm.at[wid], src_v)
    pltpu.sync_copy(dst_hbm.at[wid], dst_v)
    pltpu.sync_copy(
        y_hbm.at[pl.ds(s * STRIPE, STRIPE)],
        acc_sh.at[pl.ds(s * STRIPE, STRIPE)],
    )
    plsc.subcore_barrier()

    def body(j, carry):
        pltpu.async_copy(y_hbm.at[src_v.at[j]], rows_v, sem_a).wait()
        pltpu.sync_copy(rows_v, acc_sh.at[dst_v.at[j]], add=True)
        return carry

    lax.fori_loop(0, NCHUNK, body, 0)
    plsc.subcore_barrier()
    pltpu.sync_copy(
        acc_sh.at[pl.ds(s * STRIPE, STRIPE)],
        out_hbm.at[c, pl.ds(s * STRIPE, STRIPE)],
    )


def _dis_from(deg_ref):
    d2 = deg_ref[:, 0, 0, :]
    ones_nw = jnp.ones((NW, 1), jnp.float32)
    cnt = lax.dot_general(d2, ones_nw, (((0,), (0,)), ((), ())),
                          preferred_element_type=jnp.float32) + 1.0
    return lax.rsqrt(cnt)


def _ka_body(deg_ref, x_ref, w_ref, y_ref):
    dis = _dis_from(deg_ref)
    xw = jnp.dot(x_ref[...], w_ref[...], preferred_element_type=jnp.float32)
    y_ref[...] = xw * dis


_ka_call = pl.pallas_call(
    _ka_body,
    grid=(NRB,),
    in_specs=[
        pl.BlockSpec((NW, 1, 1, RB), lambda i: (0, i, 0, 0)),
        pl.BlockSpec((RB, DIM), lambda i: (i, 0)),
        pl.BlockSpec((DIM, DIM), lambda i: (0, 0)),
    ],
    out_specs=pl.BlockSpec((RB, DIM), lambda i: (i, 0)),
    out_shape=jax.ShapeDtypeStruct((NP, DIM), jnp.float32),
)


def _kb_body(deg_ref, y1_ref, acc_ref, w2_ref, b1_ref, h1_ref, y2_ref):
    dis = _dis_from(deg_ref)
    h1 = dis * (acc_ref[0] + acc_ref[1] - y1_ref[...]) + b1_ref[...]
    h1_ref[...] = h1
    y2_ref[...] = (
        jnp.dot(h1, w2_ref[...], preferred_element_type=jnp.float32) * dis
    )


_kb_call = pl.pallas_call(
    _kb_body,
    grid=(NRB,),
    in_specs=[
        pl.BlockSpec((NW, 1, 1, RB), lambda i: (0, i, 0, 0)),
        pl.BlockSpec((RB, DIM), lambda i: (i, 0)),
        pl.BlockSpec((NC, RB, DIM), lambda i: (0, i, 0)),
        pl.BlockSpec((DIM, DIM), lambda i: (0, 0)),
        pl.BlockSpec((1, DIM), lambda i: (0, 0)),
    ],
    out_specs=[
        pl.BlockSpec((RB, DIM), lambda i: (i, 0)),
        pl.BlockSpec((RB, DIM), lambda i: (i, 0)),
    ],
    out_shape=[
        jax.ShapeDtypeStruct((NP, DIM), jnp.float32),
        jax.ShapeDtypeStruct((NP, DIM), jnp.float32),
    ],
)


def _kc_body(deg_ref, y2_ref, acc_ref, b2_ref, h1_ref, batch_ref,
             wl1_ref, bl1_ref, wl2_ref, bl2_ref, out1_ref, out2_ref, p_scr):
    i = pl.program_id(0)
    dis = _dis_from(deg_ref)
    h2 = dis * (acc_ref[0] + acc_ref[1] - y2_ref[...]) + b2_ref[...]
    gid = lax.broadcasted_iota(jnp.int32, (N_GRAPHS, RB), 0)
    m = (gid == batch_ref[0]).astype(jnp.float32)
    pp1 = jnp.dot(m, h1_ref[...], preferred_element_type=jnp.float32)
    pp2 = jnp.dot(m, h2, preferred_element_type=jnp.float32)

    @pl.when(i == 0)
    def _():
        p_scr[:, :DIM] = pp1
        p_scr[:, DIM:] = pp2

    @pl.when(i > 0)
    def _():
        p_scr[:, :DIM] += pp1
        p_scr[:, DIM:] += pp2

    @pl.when(i == NRB - 1)
    def _():
        p = p_scr[...]
        t = jnp.dot(p, wl1_ref[...], preferred_element_type=jnp.float32)
        t = jnp.maximum(t + bl1_ref[...], 0.0)
        logits = (
            jnp.dot(t, wl2_ref[...], preferred_element_type=jnp.float32)
            + bl2_ref[...]
        )
        out1_ref[...] = logits
        col = lax.broadcasted_iota(jnp.int32, (N_GRAPHS, DIM), 1)
        neg = jnp.where(col < DIM_OUT, logits, jnp.float32(-1e30))
        mx = jnp.max(neg, axis=1, keepdims=True)
        lse = jnp.log(jnp.sum(jnp.exp(neg - mx), axis=1, keepdims=True)) + mx
        out2_ref[...] = logits - lse


_kc_call = pl.pallas_call(
    _kc_body,
    grid=(NRB,),
    in_specs=[
        pl.BlockSpec((NW, 1, 1, RB), lambda i: (0, i, 0, 0)),
        pl.BlockSpec((RB, DIM), lambda i: (i, 0)),
        pl.BlockSpec((NC, RB, DIM), lambda i: (0, i, 0)),
        pl.BlockSpec((1, DIM), lambda i: (0, 0)),
        pl.BlockSpec((RB, DIM), lambda i: (i, 0)),
        pl.BlockSpec((1, 1, RB), lambda i: (i, 0, 0)),
        pl.BlockSpec((2 * DIM, 2 * DIM), lambda i: (0, 0)),
        pl.BlockSpec((1, 2 * DIM), lambda i: (0, 0)),
        pl.BlockSpec((2 * DIM, DIM), lambda i: (0, 0)),
        pl.BlockSpec((1, DIM), lambda i: (0, 0)),
    ],
    out_specs=[
        pl.BlockSpec((N_GRAPHS, DIM), lambda i: (0, 0)),
        pl.BlockSpec((N_GRAPHS, DIM), lambda i: (0, 0)),
    ],
    out_shape=[
        jax.ShapeDtypeStruct((N_GRAPHS, DIM), jnp.float32),
        jax.ShapeDtypeStruct((N_GRAPHS, DIM), jnp.float32),
    ],
    scratch_shapes=[pltpu.VMEM((N_GRAPHS, 2 * DIM), jnp.float32)],
)


def kernel(x, edge_index, batch, W1, b1, W2, b2, Wl1, bl1, Wl2, bl2):
    src = edge_index[0].astype(jnp.int32).reshape(NW, NCHUNK, K)
    dst = edge_index[1].astype(jnp.int32).reshape(NW, NCHUNK, K)
    xp = jnp.zeros((NP, DIM), jnp.float32).at[:N_NODES].set(x)
    batch2d = (jnp.full((NP,), N_GRAPHS, jnp.int32)
               .at[:N_NODES].set(batch.astype(jnp.int32)).reshape(NRB, 1, RB))
    wl2p = jnp.zeros((2 * DIM, DIM), jnp.float32).at[:, :DIM_OUT].set(Wl2)
    bl2p = jnp.zeros((1, DIM), jnp.float32).at[0, :DIM_OUT].set(bl2)
    b1r = b1.reshape(1, DIM)
    b2r = b2.reshape(1, DIM)
    bl1r = bl1.reshape(1, 2 * DIM)

    deg_kernel = pl.kernel(
        _deg_kernel_body,
        out_type=jax.ShapeDtypeStruct((NW, 1, NP), jnp.float32),
        mesh=_make_mesh(),
        scratch_types=[
            pltpu.VMEM((NCHUNK, K), jnp.int32),
            pltpu.VMEM((NP,), jnp.float32),
        ],
        compiler_params=pltpu.CompilerParams(needs_layout_passes=False),
    )
    scatter_kernel = pl.kernel(
        _scatter_kernel_body,
        out_type=jax.ShapeDtypeStruct((NC, NP, DIM), jnp.float32),
        mesh=_make_mesh(),
        scratch_types=[
            pltpu.VMEM((NCHUNK, K), jnp.int32),
            pltpu.VMEM((NCHUNK, K), jnp.int32),
            pltpu.VMEM((K, DIM), jnp.float32),
            pltpu.VMEM_SHARED((NP, DIM), jnp.float32),
            pltpu.SemaphoreType.DMA,
        ],
    )

    deg16 = deg_kernel(dst).reshape(NW, NRB, 1, RB)
    y1 = _ka_call(deg16, xp, W1)
    acc1 = scatter_kernel(y1, src, dst)
    h1, y2 = _kb_call(deg16, y1, acc1, W2, b1r)
    acc2 = scatter_kernel(y2, src, dst)
    logits, logp = _kc_call(deg16, y2, acc2, b2r, h1, batch2d,
                            Wl1, bl1r, wl2p, bl2p)
    return (logits[:, :DIM_OUT], logp[:, :DIM_OUT])


# speedup vs baseline: 1.4673x; 1.0005x over previous
import functools
import jax
import jax.numpy as jnp
from jax import lax
from jax.experimental import pallas as pl
from jax.experimental.pallas import tpu as pltpu
from jax.experimental.pallas import tpu_sc as plsc

N_NODES = 10000
N_EDGES = 320000
N_GRAPHS = 64
DIM = 128
DIM_OUT = 10
NP = 10240

NC, NS = 2, 16
NW = NC * NS
EPW = N_EDGES // NW
K = 80
NCHUNK = EPW // K
STRIPE = NP // NS

RB = 1024
NRB = NP // RB


def _make_mesh():
    return plsc.VectorSubcoreMesh(
        core_axis_name="c", subcore_axis_name="s", num_cores=NC,
        num_subcores=NS,
    )


def _deg_kernel_body(dst_hbm, out_hbm, dst_v, hist_v):
    c = lax.axis_index("c")
    s = lax.axis_index("s")
    wid = c * NS + s
    pltpu.sync_copy(dst_hbm.at[wid], dst_v)
    zero16 = jnp.zeros((16,), jnp.float32)

    @pl.loop(0, NP // 16)
    def _zero(r):
        hist_v[pl.ds(r * 16, 16)] = zero16

    one16 = jnp.ones((16,), jnp.float32)

    @pl.loop(0, NCHUNK)
    def _hist(r):
        for cc in range(K // 16):
            idx = dst_v[r, pl.ds(cc * 16, 16)]
            plsc.addupdate_scatter(hist_v, [idx], one16)

    pltpu.sync_copy(hist_v, out_hbm.at[wid, 0])


def _scatter_kernel_body(y_hbm, src_hbm, dst_hbm, out_hbm, src_v, dst_v,
                         rows_v, acc_sh, sem_a):
    c = lax.axis_index("c")
    s = lax.axis_index("s")
    wid = c * NS + s
    pltpu.sync_copy(src_hbm.at[wid], src_v)
    pltpu.sync_copy(dst_hbm.at[wid], dst_v)
    pltpu.sync_copy(
        y_hbm.at[pl.ds(s * STRIPE, STRIPE)],
        acc_sh.at[pl.ds(s * STRIPE, STRIPE)],
    )
    plsc.subcore_barrier()

    def body(j, carry):
        pltpu.async_copy(y_hbm.at[src_v.at[j]], rows_v, sem_a).wait()
        pltpu.sync_copy(rows_v, acc_sh.at[dst_v.at[j]], add=True)
        return carry

    lax.fori_loop(0, NCHUNK, body, 0)
    plsc.subcore_barrier()
    pltpu.sync_copy(
        acc_sh.at[pl.ds(s * STRIPE, STRIPE)],
        out_hbm.at[c, pl.ds(s * STRIPE, STRIPE)],
    )


def _dis_from(deg_ref):
    d2 = deg_ref[:, 0, 0, :]
    ones_nw = jnp.ones((NW, 1), jnp.float32)
    cnt = lax.dot_general(d2, ones_nw, (((0,), (0,)), ((), ())),
                          preferred_element_type=jnp.float32) + 1.0
    return lax.rsqrt(cnt)


def _ka_body(deg_ref, x_ref, w_ref, y_ref):
    dis = _dis_from(deg_ref)
    xw = jnp.dot(x_ref[...], w_ref[...], preferred_element_type=jnp.float32)
    y_ref[...] = xw * dis


_ka_call = pl.pallas_call(
    _ka_body,
    grid=(NRB,),
    in_specs=[
        pl.BlockSpec((NW, 1, 1, RB), lambda i: (0, i, 0, 0)),
        pl.BlockSpec((RB, DIM), lambda i: (i, 0)),
        pl.BlockSpec((DIM, DIM), lambda i: (0, 0)),
    ],
    out_specs=pl.BlockSpec((RB, DIM), lambda i: (i, 0)),
    out_shape=jax.ShapeDtypeStruct((NP, DIM), jnp.float32),
)


def _kb_body(deg_ref, y1_ref, acc_ref, w2_ref, b1_ref, h1_ref, y2_ref):
    dis = _dis_from(deg_ref)
    h1 = dis * (acc_ref[0] + acc_ref[1] - y1_ref[...]) + b1_ref[...]
    h1_ref[...] = h1
    y2_ref[...] = (
        jnp.dot(h1, w2_ref[...], preferred_element_type=jnp.float32) * dis
    )


_kb_call = pl.pallas_call(
    _kb_body,
    grid=(NRB,),
    in_specs=[
        pl.BlockSpec((NW, 1, 1, RB), lambda i: (0, i, 0, 0)),
        pl.BlockSpec((RB, DIM), lambda i: (i, 0)),
        pl.BlockSpec((NC, RB, DIM), lambda i: (0, i, 0)),
        pl.BlockSpec((DIM, DIM), lambda i: (0, 0)),
        pl.BlockSpec((1, DIM), lambda i: (0, 0)),
    ],
    out_specs=[
        pl.BlockSpec((RB, DIM), lambda i: (i, 0)),
        pl.BlockSpec((RB, DIM), lambda i: (i, 0)),
    ],
    out_shape=[
        jax.ShapeDtypeStruct((NP, DIM), jnp.float32),
        jax.ShapeDtypeStruct((NP, DIM), jnp.float32),
    ],
)


def _kc_body(deg_ref, y2_ref, acc_ref, b2_ref, h1_ref, batch_ref,
             wl1_ref, bl1_ref, wl2_ref, bl2_ref, out1_ref, out2_ref, p_scr):
    i = pl.program_id(0)
    dis = _dis_from(deg_ref)
    h2 = dis * (acc_ref[0] + acc_ref[1] - y2_ref[...]) + b2_ref[...]
    gid = lax.broadcasted_iota(jnp.int32, (N_GRAPHS, RB), 0)
    m = (gid == batch_ref[0]).astype(jnp.float32)
    pp1 = jnp.dot(m, h1_ref[...], preferred_element_type=jnp.float32)
    pp2 = jnp.dot(m, h2, preferred_element_type=jnp.float32)

    @pl.when(i == 0)
    def _():
        p_scr[:, :DIM] = pp1
        p_scr[:, DIM:] = pp2

    @pl.when(i > 0)
    def _():
        p_scr[:, :DIM] += pp1
        p_scr[:, DIM:] += pp2

    @pl.when(i == NRB - 1)
    def _():
        p = p_scr[...]
        t = jnp.dot(p, wl1_ref[...], preferred_element_type=jnp.float32)
        t = jnp.maximum(t + bl1_ref[...], 0.0)
        logits = (
            jnp.dot(t, wl2_ref[...], preferred_element_type=jnp.float32)
            + bl2_ref[...]
        )
        out1_ref[...] = logits
        col = lax.broadcasted_iota(jnp.int32, (N_GRAPHS, DIM), 1)
        neg = jnp.where(col < DIM_OUT, logits, jnp.float32(-1e30))
        mx = jnp.max(neg, axis=1, keepdims=True)
        lse = jnp.log(jnp.sum(jnp.exp(neg - mx), axis=1, keepdims=True)) + mx
        out2_ref[...] = logits - lse


_kc_call = pl.pallas_call(
    _kc_body,
    grid=(NRB,),
    in_specs=[
        pl.BlockSpec((NW, 1, 1, RB), lambda i: (0, i, 0, 0)),
        pl.BlockSpec((RB, DIM), lambda i: (i, 0)),
        pl.BlockSpec((NC, RB, DIM), lambda i: (0, i, 0)),
        pl.BlockSpec((1, DIM), lambda i: (0, 0)),
        pl.BlockSpec((RB, DIM), lambda i: (i, 0)),
        pl.BlockSpec((1, 1, RB), lambda i: (i, 0, 0)),
        pl.BlockSpec((2 * DIM, 2 * DIM), lambda i: (0, 0)),
        pl.BlockSpec((1, 2 * DIM), lambda i: (0, 0)),
        pl.BlockSpec((2 * DIM, DIM), lambda i: (0, 0)),
        pl.BlockSpec((1, DIM), lambda i: (0, 0)),
    ],
    out_specs=[
        pl.BlockSpec((N_GRAPHS, DIM), lambda i: (0, 0)),
        pl.BlockSpec((N_GRAPHS, DIM), lambda i: (0, 0)),
    ],
    out_shape=[
        jax.ShapeDtypeStruct((N_GRAPHS, DIM), jnp.float32),
        jax.ShapeDtypeStruct((N_GRAPHS, DIM), jnp.float32),
    ],
    scratch_shapes=[pltpu.VMEM((N_GRAPHS, 2 * DIM), jnp.float32)],
)


def kernel(x, edge_index, batch, W1, b1, W2, b2, Wl1, bl1, Wl2, bl2):
    src = edge_index[0].astype(jnp.int32).reshape(NW, NCHUNK, K)
    dst = edge_index[1].astype(jnp.int32).reshape(NW, NCHUNK, K)
    xp = jnp.zeros((NP, DIM), jnp.float32).at[:N_NODES].set(x)
    batch2d = (jnp.full((NP,), N_GRAPHS, jnp.int32)
               .at[:N_NODES].set(batch.astype(jnp.int32)).reshape(NRB, 1, RB))
    wl2p = jnp.zeros((2 * DIM, DIM), jnp.float32).at[:, :DIM_OUT].set(Wl2)
    bl2p = jnp.zeros((1, DIM), jnp.float32).at[0, :DIM_OUT].set(bl2)
    b1r = b1.reshape(1, DIM)
    b2r = b2.reshape(1, DIM)
    bl1r = bl1.reshape(1, 2 * DIM)

    deg_kernel = pl.kernel(
        _deg_kernel_body,
        out_type=jax.ShapeDtypeStruct((NW, 1, NP), jnp.float32),
        mesh=_make_mesh(),
        scratch_types=[
            pltpu.VMEM((NCHUNK, K), jnp.int32),
            pltpu.VMEM((NP,), jnp.float32),
        ],
        compiler_params=pltpu.CompilerParams(needs_layout_passes=False),
    )
    scatter_kernel = pl.kernel(
        _scatter_kernel_body,
        out_type=jax.ShapeDtypeStruct((NC, NP, DIM), jnp.float32),
        mesh=_make_mesh(),
        scratch_types=[
            pltpu.VMEM((NCHUNK, K), jnp.int32),
            pltpu.VMEM((NCHUNK, K), jnp.int32),
            pltpu.VMEM((K, DIM), jnp.float32),
            pltpu.VMEM_SHARED((NP, DIM), jnp.float32),
            pltpu.SemaphoreType.DMA,
        ],
    )

    deg16 = deg_kernel(dst).reshape(NW, NRB, 1, RB)
    y1 = _ka_call(deg16, xp, W1)
    acc1 = scatter_kernel(y1, src, dst)
    h1, y2 = _kb_call(deg16, y1, acc1, W2, b1r)
    acc2 = scatter_kernel(y2, src, dst)
    logits, logp = _kc_call(deg16, y2, acc2, b2r, h1, batch2d,
                            Wl1, bl1r, wl2p, bl2p)
    return (logits[:, :DIM_OUT], logp[:, :DIM_OUT])
